# baseline (speedup 1.0000x reference)
import functools

import jax
import jax.numpy as jnp
from jax import lax
from jax.experimental import pallas as pl
from jax.experimental.pallas import tpu as pltpu
from jax.experimental.pallas import tpu_sc as plsc

_N = 10000
_E = 320000
_D = 128
_K = 5000
_NACC = 10240
_KACC = 5120
_CH = 128
_NCH = 80
_EPW = _CH * _NCH
_EPAD = 32 * _EPW


_DH = 64


def _seg_add_body(nacc, weighted, *refs):
    if weighted:
        (table_h, srcp_h, dstp_h, wp_h, out_h,
         sidx, didx, wv, rows0, rows1, acc, sem0, sem1) = refs
    else:
        (table_h, srcp_h, dstp_h, out_h,
         sidx, didx, rows0, rows1, acc, sem0, sem1) = refs
    cid = lax.axis_index("c")
    sid = lax.axis_index("s")
    wid = sid * 2 + cid
    rps = nacc // 16

    pltpu.sync_copy(srcp_h.at[wid], sidx)
    pltpu.sync_copy(dstp_h.at[wid], didx)
    if weighted:
        pltpu.sync_copy(wp_h.at[wid], wv.at[pl.ds(0, _EPW)])

    zv = jnp.zeros((16,), jnp.float32)

    def zrow(i, c):
        for jj in range(_DH // 16):
            rows0[i, pl.ds(jj * 16, 16)] = zv
        return c

    lax.fori_loop(0, _CH, zrow, 0)

    def zacc(t, c):
        pltpu.sync_copy(rows0.at[pl.ds(0, 64)],
                        acc.at[pl.ds(sid * rps + t * 64, 64)])
        return c

    lax.fori_loop(0, rps // 64, zacc, 0)
    plsc.subcore_barrier()

    pltpu.async_copy(table_h.at[sidx.at[0]], rows0, sem0)
    pltpu.async_copy(table_h.at[sidx.at[1]], rows1, sem1)

    def chunk(i, c):
        for b in range(2):
            rows = rows0 if b == 0 else rows1
            sem = sem0 if b == 0 else sem1
            j = i * 2 + b
            pltpu.make_async_copy(table_h.at[sidx.at[j]], rows, sem).wait()
            if weighted:

                def rowscale(r, cc):
                    wr = wv[pl.ds(j * _CH + r, 16)][0]
                    for jj in range(_DH // 16):
                        rows[r, pl.ds(jj * 16, 16)] = (
                            rows[r, pl.ds(jj * 16, 16)] * wr)
                    return cc

                lax.fori_loop(0, _CH, rowscale, 0)
            pltpu.sync_copy(rows, acc.at[didx.at[j]], add=True)
            nj = j + 2

            @pl.when(nj < _NCH)
            def _():
                pltpu.async_copy(table_h.at[sidx.at[nj]], rows, sem)
        return c

    lax.fori_loop(0, _NCH // 2, chunk, 0)
    plsc.subcore_barrier()
    pltpu.sync_copy(acc.at[pl.ds(sid * rps, rps)],
                    out_h.at[cid, pl.ds(sid * rps, rps)])


@functools.partial(jax.jit, static_argnames=("nacc",))
def _sc_seg_add_h(table_h64, srcp3, dstp3, wp3=None, *, nacc):
    weighted = wp3 is not None
    mesh = plsc.VectorSubcoreMesh(core_axis_name="c", subcore_axis_name="s")
    scratch = [
        pltpu.VMEM((_NCH, _CH), jnp.int32),
        pltpu.VMEM((_NCH, _CH), jnp.int32),
    ]
    if weighted:
        scratch.append(pltpu.VMEM((_EPW + 16,), jnp.float32))
    scratch += [
        pltpu.VMEM((_CH, _DH), jnp.float32),
        pltpu.VMEM((_CH, _DH), jnp.float32),
        pltpu.VMEM_SHARED((nacc, _DH), jnp.float32),
        pltpu.SemaphoreType.DMA,
        pltpu.SemaphoreType.DMA,
    ]
    body = functools.partial(_seg_add_body, nacc, weighted)
    args = (table_h64, srcp3, dstp3) + ((wp3,) if weighted else ())
    return pl.kernel(
        body,
        out_type=jax.ShapeDtypeStruct((2, nacc, _DH), jnp.float32),
        mesh=mesh,
        scratch_types=scratch,
        compiler_params=pltpu.CompilerParams(use_tc_tiling_on_sc=False),
    )(*args)


def _sc_seg_add(table, srcp3, dstp3, wp3=None, *, nacc):
    lo = _sc_seg_add_h(table[:, :_DH], srcp3, dstp3, wp3, nacc=nacc)
    hi = _sc_seg_add_h(table[:, _DH:], srcp3, dstp3, wp3, nacc=nacc)
    return lo, hi


def _tc(body, out_shape, *args):
    return pl.pallas_call(body, out_shape=out_shape)(*args)


def _tc1_body(x_ref, w_ref, indeg_ref, m1_ref, dinv_ref):
    dinv = lax.rsqrt(indeg_ref[...] + 1.0)
    dinv_ref[...] = dinv
    m1_ref[...] = jnp.dot(x_ref[...], w_ref[...],
                          preferred_element_type=jnp.float32) * dinv


def _acc_cat(lo_ref, hi_ref, n):
    return jnp.concatenate([lo_ref[0, :n, :] + lo_ref[1, :n, :],
                            hi_ref[0, :n, :] + hi_ref[1, :n, :]], axis=1)


def _tc2_body(acclo_ref, acchi_ref, m1_ref, dinv_ref, b_ref,
              xr_ref, h1_ref, g1_ref):
    s = _acc_cat(acclo_ref, acchi_ref, _N) + m1_ref[...]
    h1 = dinv_ref[...] * s + b_ref[...]
    h1_ref[...] = h1
    xr = jax.nn.relu(h1)
    xr_ref[...] = xr
    g1_ref[...] = jnp.concatenate(
        [jnp.max(xr, axis=0, keepdims=True),
         jnp.sum(xr, axis=0, keepdims=True) / _N], axis=1)


def _tc3_body(h1_ref, xr_ref, w1_ref, b1_ref, w2_ref, b2_ref,
              g0_ref, p1_ref):
    w1 = w1_ref[...]
    b1 = b1_ref[...]
    w2 = w2_ref[...]
    b2 = b2_ref[...]
    s0 = jnp.sum(jax.nn.relu(jnp.dot(h1_ref[...], w1,
                                     preferred_element_type=jnp.float32) + b1),
                 axis=0, keepdims=True)
    g0_ref[...] = jnp.dot(s0, w2, preferred_element_type=jnp.float32) + _N * b2
    s1 = jnp.sum(jax.nn.relu(jnp.dot(xr_ref[...], w1,
                                     preferred_element_type=jnp.float32) + b1),
                 axis=0, keepdims=True)
    p1_ref[...] = jnp.dot(s1, w2, preferred_element_type=jnp.float32) + _N * b2


def _tc4_body(xq_ref, xr_ref, wlin_ref, v_ref, c0_ref, abp_ref):
    a = jnp.dot(xq_ref[...], wlin_ref[...],
                preferred_element_type=jnp.float32) + c0_ref[0, 0]
    b = jnp.dot(xr_ref[...], v_ref[...], preferred_element_type=jnp.float32)
    ps = jnp.exp(jax.nn.leaky_relu(a + b, 0.2))
    abp_ref[...] = jnp.concatenate([a, b, ps], axis=1)


def _tc5_body(xnlo_ref, xnhi_ref, xr_ref, sc_ref, lw_ref,
              xn_ref, abc_ref):
    xn = (_acc_cat(xnlo_ref, xnhi_ref, _N) + xr_ref[...] * sc_ref[...])
    xn_ref[...] = xn
    abc_ref[...] = jnp.dot(xn, lw_ref[...], preferred_element_type=jnp.float32)


def _tc6_body(x1_ref, w_ref, deg2_ref, m2_ref, dinv2_ref):
    dinv2 = lax.rsqrt(deg2_ref[...])
    dinv2_ref[...] = dinv2
    m2 = jnp.dot(x1_ref[...], w_ref[...],
                 preferred_element_type=jnp.float32) * dinv2
    m2_ref[pl.ds(0, _K), :] = m2
    m2_ref[pl.ds(_K, _KACC - _K), :] = jnp.zeros((_KACC - _K, _D), jnp.float32)


def _tc7_body(acclo_ref, acchi_ref, m2_ref, dinv2_ref, b_ref,
              nw1_ref, nb1_ref, nw2_ref, nb2_ref,
              pw1_ref, pb1_ref, pw2_ref, pb2_ref,
              qw1_ref, qb1_ref, qw2_ref, qb2_ref,
              e1_ref, e2_ref, tf_ref, g1_ref,
              out0_ref, g11_ref, g12_ref, p2_ref):
    x1g = (dinv2_ref[...]
           * (_acc_cat(acclo_ref, acchi_ref, _K) + 2.0 * m2_ref[:_K, :])
           + b_ref[...])
    nz1 = jax.nn.relu(jnp.dot(x1g, nw1_ref[...],
                              preferred_element_type=jnp.float32) + nb1_ref[...])
    nz = jnp.dot(nz1, nw2_ref[...], preferred_element_type=jnp.float32) + nb2_ref[...]
    mean = nz[:, :_D]
    logvar = jnp.clip(nz[:, _D:], -30.0, 20.0)
    std = jnp.exp(0.5 * logvar)
    tf = tf_ref[0, 0]
    keep = tf > 0
    x11 = jnp.where(keep, (1.0 - tf) * x1g + tf * (mean + std * e1_ref[...]), x1g)
    x12 = jnp.where(keep, (1.0 - tf) * x1g + tf * (mean + std * e2_ref[...]), x1g)
    pw1 = pw1_ref[...]
    pb1 = pb1_ref[...]
    pw2 = pw2_ref[...]
    pb2 = pb2_ref[...]

    def psum(z, w1, b1, w2, b2):
        s = jnp.sum(jax.nn.relu(jnp.dot(z, w1,
                                        preferred_element_type=jnp.float32) + b1),
                    axis=0, keepdims=True)
        return jnp.dot(s, w2, preferred_element_type=jnp.float32) + _K * b2

    g11_ref[...] = psum(x11, pw1, pb1, pw2, pb2)
    g12_ref[...] = psum(x12, pw1, pb1, pw2, pb2)
    x1r = jax.nn.relu(x1g)
    g2 = jnp.concatenate([jnp.max(x1r, axis=0, keepdims=True),
                          jnp.sum(x1r, axis=0, keepdims=True) / _K], axis=1)
    out0_ref[...] = g1_ref[...] + g2
    p2_ref[...] = psum(x1r, qw1_ref[...], qb1_ref[...], qw2_ref[...], qb2_ref[...])


def _seg_sum(v, ids, n):
    return jax.ops.segment_sum(v, ids, num_segments=n)


def _pad_edges(v, fill):
    return jnp.concatenate(
        [v, jnp.full((_EPAD - _E,), fill, v.dtype)]).reshape(32, _NCH, _CH)


def kernel(x, params, edge_index, batch, tradeoff):
    p = params
    f32 = jnp.float32
    src = edge_index[0]
    dst = edge_index[1]

    srcp3 = _pad_edges(src, 0)
    dstp3 = _pad_edges(dst, _NACC - 1)

    indeg = _seg_sum(jnp.ones((_E,), f32), dst, _N)
    indeg_c = indeg[:, None]

    m1, dinv_c = _tc(
        _tc1_body,
        [jax.ShapeDtypeStruct((_N, _D), f32),
         jax.ShapeDtypeStruct((_N, 1), f32)],
        x, p['conv1_w'], indeg_c)

    acc1lo, acc1hi = _sc_seg_add(m1, srcp3, dstp3, nacc=_NACC)

    xr, h1, g1 = _tc(
        _tc2_body,
        [jax.ShapeDtypeStruct((_N, _D), f32),
         jax.ShapeDtypeStruct((_N, _D), f32),
         jax.ShapeDtypeStruct((1, 2 * _D), f32)],
        acc1lo, acc1hi, m1, dinv_c, p['conv1_b'][None, :])

    g0, proj_1 = _tc(
        _tc3_body,
        [jax.ShapeDtypeStruct((1, _D), f32),
         jax.ShapeDtypeStruct((1, _D), f32)],
        h1, xr, p['p1_w1'], p['p1_b1'][None, :], p['p1_w2'], p['p1_b2'][None, :])

    accmax = jax.ops.segment_max(xr[src], dst, num_segments=_N)
    x_q_raw = jnp.maximum(accmax, xr)

    u = p['pool_att_w'][:_D, 0]
    v = p['pool_att_w'][_D:, 0]
    wlin = (p['pool_lin_w'] @ u)[:, None]
    c0 = (p['pool_lin_b'] @ u + p['pool_att_b'][0])[None, None]

    abp = _tc(
        _tc4_body,
        jax.ShapeDtypeStruct((_N, 3), f32),
        x_q_raw, xr, wlin, v[:, None], c0)

    a_n = abp[:, 0]
    b_n = abp[:, 1]
    ps = abp[:, 2]
    pe = jnp.exp(jax.nn.leaky_relu(a_n[dst] + b_n[src], 0.2))
    qsum = _seg_sum(pe, dst, _N)
    rden = 1.0 / (qsum + ps + 1e-16)
    score_e = pe * rden[dst]

    wp3 = _pad_edges(score_e, 0.0).reshape(32, _EPW)
    xnlo, xnhi = _sc_seg_add(xr, srcp3, dstp3, wp3, nacc=_NACC)

    lw = jnp.concatenate([p['le_w1'], p['le_w2'], p['le_w3']], axis=1)
    x_new, abc = _tc(
        _tc5_body,
        [jax.ShapeDtypeStruct((_N, _D), f32),
         jax.ShapeDtypeStruct((_N, 3), f32)],
        xnlo, xnhi, xr, (ps * rden)[:, None], lw)

    a2 = abc[:, 0]
    bb2 = abc[:, 1]
    sumbb = _seg_sum(bb2[src], dst, _N) + bb2
    fit = jax.nn.sigmoid((indeg + 1.0) * a2 - sumbb + abc[:, 2] + p['le_b'][0])

    fitvals, perm = lax.top_k(fit, _K)
    x1 = x_new[perm] * fitvals[:, None]
    inv = jnp.full((_N,), -1, jnp.int32).at[perm].set(
        jnp.arange(_K, dtype=jnp.int32))
    ns = inv[src]
    nd = inv[dst]
    valid = (ns >= 0) & (nd >= 0)
    nsv = jnp.where(valid, ns, _K)
    ndv = jnp.where(valid, nd, _KACC - 1)
    deg2 = 2.0 + _seg_sum(valid.astype(f32), jnp.where(valid, nd, _K), _K + 1)[:_K]

    nsp3 = _pad_edges(nsv, _K)
    ndp3 = _pad_edges(ndv, _KACC - 1)

    m2pad, dinv2_c = _tc(
        _tc6_body,
        [jax.ShapeDtypeStruct((_KACC, _D), f32),
         jax.ShapeDtypeStruct((_K, 1), f32)],
        x1, p['conv2_w'], deg2[:, None])

    acc2lo, acc2hi = _sc_seg_add(m2pad, nsp3, ndp3, nacc=_KACC)

    kk = jax.random.key(1234)
    e1 = jax.random.normal(jax.random.fold_in(kk, 1), (_K, _D), f32)
    e2 = jax.random.normal(jax.random.fold_in(kk, 2), (_K, _D), f32)
    tf = jnp.asarray(tradeoff, f32)[None, None]

    out0, g1_1, g1_2, proj_2 = _tc(
        _tc7_body,
        [jax.ShapeDtypeStruct((1, 2 * _D), f32),
         jax.ShapeDtypeStruct((1, _D), f32),
         jax.ShapeDtypeStruct((1, _D), f32),
         jax.ShapeDtypeStruct((1, _D), f32)],
        acc2lo, acc2hi, m2pad, dinv2_c, p['conv2_b'][None, :],
        p['n_w1'], p['n_b1'][None, :], p['n_w2'], p['n_b2'][None, :],
        p['p1_w1'], p['p1_b1'][None, :], p['p1_w2'], p['p1_b2'][None, :],
        p['p2_w1'], p['p2_b1'][None, :], p['p2_w2'], p['p2_b2'][None, :],
        e1, e2, tf, g1)

    return (out0, proj_1, proj_2, g0, g0, g1_1, g1_2)
